# Initial kernel scaffold; baseline (speedup 1.0000x reference)
import jax
import jax.numpy as jnp
from jax.experimental import pallas as pl


def kernel(x, edge_index, edge_attr, params, Wlin, blin):
    raise NotImplementedError("write your pallas kernel here")


# speedup vs baseline: 7.4840x; 7.4840x over previous
import functools

import jax
import jax.numpy as jnp
from jax import lax
from jax.experimental import pallas as pl
from jax.experimental.pallas import tpu as pltpu
from jax.experimental.pallas import tpu_sc as plsc

N = 10000
E = 320000
D = 128
H = 16
DE = 16

NC = 2
NS = 16
NW = NC * NS
EPW = E // NW
C = 80
NCH = EPW // C

_mesh = plsc.VectorSubcoreMesh(
    core_axis_name="c", subcore_axis_name="s", num_cores=NC, num_subcores=NS)

f32 = jnp.float32


def _edge_emb_body(ea_ref, we_ref, out_ref):
    out_ref[0] = jnp.dot(ea_ref[...], we_ref[0], preferred_element_type=f32)


def _edge_emb(edge_attr, we3):
    EB = 4000
    return pl.pallas_call(
        _edge_emb_body,
        grid=(3, E // EB),
        in_specs=[
            pl.BlockSpec((EB, DE), lambda l, i: (i, 0)),
            pl.BlockSpec((1, DE, H), lambda l, i: (l, 0, 0)),
        ],
        out_specs=pl.BlockSpec((1, EB, H), lambda l, i: (l, i, 0)),
        out_shape=jax.ShapeDtypeStruct((3, E, H), f32),
    )(edge_attr, we3)


def _proj_body(x_ref, wl_ref, bl_ref, wr_ref, br_ref, xl_ref, xr_ref):
    xv = x_ref[...]
    xl_ref[...] = jnp.dot(xv, wl_ref[...], preferred_element_type=f32) + bl_ref[0]
    xr_ref[...] = jnp.dot(xv, wr_ref[...], preferred_element_type=f32) + br_ref[0]


def _proj(x, wl, bl, wr, br):
    NB = 2000
    din = x.shape[1]
    return pl.pallas_call(
        _proj_body,
        grid=(N // NB,),
        in_specs=[
            pl.BlockSpec((NB, din), lambda i: (i, 0)),
            pl.BlockSpec((din, H), lambda i: (0, 0)),
            pl.BlockSpec((1, H), lambda i: (0, 0)),
            pl.BlockSpec((din, H), lambda i: (0, 0)),
            pl.BlockSpec((1, H), lambda i: (0, 0)),
        ],
        out_specs=[
            pl.BlockSpec((NB, H), lambda i: (i, 0)),
            pl.BlockSpec((NB, H), lambda i: (i, 0)),
        ],
        out_shape=[
            jax.ShapeDtypeStruct((N, H), f32),
            jax.ShapeDtypeStruct((N, H), f32),
        ],
    )(x, wl, bl, wr, br)


def _combine_proj_body(a0_ref, a1_ref, d0_ref, d1_ref, bias_ref,
                       wl_ref, bl_ref, wr_ref, br_ref, xl_ref, xr_ref):
    acc = a0_ref[...] + a1_ref[...]
    den = d0_ref[...] + d1_ref[...] + 1e-16
    h = acc / den + bias_ref[0]
    xl_ref[...] = jnp.dot(h, wl_ref[...], preferred_element_type=f32) + bl_ref[0]
    xr_ref[...] = jnp.dot(h, wr_ref[...], preferred_element_type=f32) + br_ref[0]


def _combine_proj(a0, a1, d0, d1, bias, wl, bl, wr, br):
    NB = 2000
    return pl.pallas_call(
        _combine_proj_body,
        grid=(N // NB,),
        in_specs=[
            pl.BlockSpec((NB, H), lambda i: (i, 0)),
            pl.BlockSpec((NB, H), lambda i: (i, 0)),
            pl.BlockSpec((NB, 1), lambda i: (i, 0)),
            pl.BlockSpec((NB, 1), lambda i: (i, 0)),
            pl.BlockSpec((1, H), lambda i: (0, 0)),
            pl.BlockSpec((H, H), lambda i: (0, 0)),
            pl.BlockSpec((1, H), lambda i: (0, 0)),
            pl.BlockSpec((H, H), lambda i: (0, 0)),
            pl.BlockSpec((1, H), lambda i: (0, 0)),
        ],
        out_specs=[
            pl.BlockSpec((NB, H), lambda i: (i, 0)),
            pl.BlockSpec((NB, H), lambda i: (i, 0)),
        ],
        out_shape=[
            jax.ShapeDtypeStruct((N, H), f32),
            jax.ShapeDtypeStruct((N, H), f32),
        ],
    )(a0, a1, d0, d1, bias, wl, bl, wr, br)


def _final_body(a0_ref, a1_ref, d0_ref, d1_ref, bias_ref, wlin_ref, blin_ref,
                out_ref):
    i = pl.program_id(0)
    acc = a0_ref[...] + a1_ref[...]
    den = d0_ref[...] + d1_ref[...] + 1e-16
    h = acc / den + bias_ref[0]
    part = jnp.sum(jnp.dot(h, wlin_ref[...], preferred_element_type=f32))

    @pl.when(i == 0)
    def _():
        out_ref[...] = jnp.zeros((1, 1), f32)

    out_ref[...] += jnp.reshape(part / N, (1, 1))

    @pl.when(i == pl.num_programs(0) - 1)
    def _():
        out_ref[...] += blin_ref[...]


def _final(a0, a1, d0, d1, bias, wlin, blin):
    NB = 2000
    return pl.pallas_call(
        _final_body,
        grid=(N // NB,),
        in_specs=[
            pl.BlockSpec((NB, H), lambda i: (i, 0)),
            pl.BlockSpec((NB, H), lambda i: (i, 0)),
            pl.BlockSpec((NB, 1), lambda i: (i, 0)),
            pl.BlockSpec((NB, 1), lambda i: (i, 0)),
            pl.BlockSpec((1, H), lambda i: (0, 0)),
            pl.BlockSpec((H, 1), lambda i: (0, 0)),
            pl.BlockSpec((1, 1), lambda i: (0, 0)),
        ],
        out_specs=pl.BlockSpec((1, 1), lambda i: (0, 0)),
        out_shape=jax.ShapeDtypeStruct((1, 1), f32),
    )(a0, a1, d0, d1, bias, wlin, blin)


def _pass1_body(src_hbm, dst_hbm, xl_hbm, xr_hbm, el_hbm, att_hbm, zn_hbm,
                alpha_hbm, ssum0_hbm, ssum1_hbm, cnt0_hbm, cnt1_hbm,
                src_v, dst_v, xlv, xrv, ev, av, onesv, attv,
                ssum_sh, cnt_sh, sem1, sem2):
    c = lax.axis_index("c")
    s = lax.axis_index("s")
    wid = s * NC + c
    base0 = wid * EPW

    pltpu.sync_copy(att_hbm, attv)
    ones16 = jnp.full((16,), 1.0, f32)
    for b in range(C // 16):
        onesv[pl.ds(16 * b, 16)] = ones16

    @pl.when(s == 0)
    def _():
        pltpu.sync_copy(zn_hbm, ssum_sh)
        pltpu.sync_copy(zn_hbm, cnt_sh)

    plsc.subcore_barrier()

    attvec = attv[...]
    attks = [attvec[k] for k in range(H)]
    iota16 = lax.iota(jnp.int32, 16)
    kvecs = [jnp.full((16,), k, jnp.int32) for k in range(H)]

    def chunk(j, carry):
        base = base0 + j * C
        pltpu.sync_copy(src_hbm.at[pl.ds(base, C)], src_v)
        pltpu.sync_copy(dst_hbm.at[pl.ds(base, C)], dst_v)
        cp1 = pltpu.async_copy(xl_hbm.at[src_v], xlv, sem1)
        cp2 = pltpu.async_copy(xr_hbm.at[dst_v], xrv, sem2)
        pltpu.sync_copy(el_hbm.at[pl.ds(base, C)], ev)
        cp1.wait()
        cp2.wait()

        def group(b, carry2):
            ivec = iota16 + b * 16
            acc = jnp.zeros((16,), f32)
            for k in range(H):
                z = (plsc.load_gather(xlv, [ivec, kvecs[k]])
                     + plsc.load_gather(xrv, [ivec, kvecs[k]])
                     + plsc.load_gather(ev, [ivec, kvecs[k]]))
                m = jnp.maximum(z, 0.2 * z)
                acc = acc + m * attks[k]
            av[pl.ds(b * 16, 16)] = acc
            return carry2

        lax.fori_loop(0, C // 16, group, 0)
        pltpu.sync_copy(av, alpha_hbm.at[pl.ds(base, C)])
        pltpu.sync_copy(av, ssum_sh.at[dst_v], add=True)
        pltpu.sync_copy(onesv, cnt_sh.at[dst_v], add=True)
        return carry

    lax.fori_loop(0, NCH, chunk, 0)

    plsc.subcore_barrier()

    @pl.when((s == 0) & (c == 0))
    def _():
        pltpu.sync_copy(ssum_sh, ssum0_hbm)
        pltpu.sync_copy(cnt_sh, cnt0_hbm)

    @pl.when((s == 0) & (c == 1))
    def _():
        pltpu.sync_copy(ssum_sh, ssum1_hbm)
        pltpu.sync_copy(cnt_sh, cnt1_hbm)


_sc_params = pltpu.CompilerParams(
    needs_layout_passes=False, use_tc_tiling_on_sc=False)

_pass1 = functools.partial(
    pl.kernel,
    compiler_params=_sc_params,
    out_type=[
        jax.ShapeDtypeStruct((E,), f32),
        jax.ShapeDtypeStruct((N,), f32),
        jax.ShapeDtypeStruct((N,), f32),
        jax.ShapeDtypeStruct((N,), f32),
        jax.ShapeDtypeStruct((N,), f32),
    ],
    mesh=_mesh,
    scratch_types=[
        pltpu.VMEM((C,), jnp.int32),
        pltpu.VMEM((C,), jnp.int32),
        pltpu.VMEM((C, H), f32),
        pltpu.VMEM((C, H), f32),
        pltpu.VMEM((C, H), f32),
        pltpu.VMEM((C,), f32),
        pltpu.VMEM((C,), f32),
        pltpu.VMEM((16,), f32),
        pltpu.VMEM_SHARED((N,), f32),
        pltpu.VMEM_SHARED((N,), f32),
        pltpu.SemaphoreType.DMA,
        pltpu.SemaphoreType.DMA,
    ],
)(_pass1_body)


def _pass2_body(src_hbm, dst_hbm, xl_hbm, alpha_hbm,
                ssum0_hbm, ssum1_hbm, cnt0_hbm, cnt1_hbm, zn_hbm, znh_hbm,
                acc0_hbm, acc1_hbm, den0_hbm, den1_hbm,
                src_v, dst_v, xlv, av, exv, rowv,
                t0v, t1v, c0v, c1v, shift_v,
                acc_sh, den_sh, sem1):
    c = lax.axis_index("c")
    s = lax.axis_index("s")
    wid = s * NC + c
    base0 = wid * EPW

    pltpu.sync_copy(ssum0_hbm, t0v)
    pltpu.sync_copy(ssum1_hbm, t1v)
    pltpu.sync_copy(cnt0_hbm, c0v)
    pltpu.sync_copy(cnt1_hbm, c1v)

    @pl.when(s == 0)
    def _():
        pltpu.sync_copy(znh_hbm, acc_sh)
        pltpu.sync_copy(zn_hbm, den_sh)

    def sbody(i, carry):
        sl = pl.ds(i * 16, 16)
        tot = t0v[sl] + t1v[sl]
        cnt = jnp.maximum(c0v[sl] + c1v[sl], 1.0)
        shift_v[sl] = tot / cnt
        return carry

    lax.fori_loop(0, N // 16, sbody, 0)

    plsc.subcore_barrier()

    def chunk(j, carry):
        base = base0 + j * C
        pltpu.sync_copy(src_hbm.at[pl.ds(base, C)], src_v)
        pltpu.sync_copy(dst_hbm.at[pl.ds(base, C)], dst_v)
        cp1 = pltpu.async_copy(xl_hbm.at[src_v], xlv, sem1)
        pltpu.sync_copy(alpha_hbm.at[pl.ds(base, C)], av)

        def gbody(b, carry2):
            sl = pl.ds(b * 16, 16)
            dvec = dst_v[sl]
            svec = plsc.load_gather(shift_v, [dvec])
            exv[sl] = jnp.exp(av[sl] - svec)
            return carry2

        lax.fori_loop(0, C // 16, gbody, 0)
        cp1.wait()
        for b in range(C // 16):
            exvec = exv[pl.ds(b * 16, 16)]
            for t in range(16):
                i = b * 16 + t
                rowv[i, :] = xlv[i, :] * exvec[t]
        pltpu.sync_copy(exv, den_sh.at[dst_v], add=True)
        pltpu.sync_copy(rowv, acc_sh.at[dst_v], add=True)
        return carry

    lax.fori_loop(0, NCH, chunk, 0)

    plsc.subcore_barrier()

    @pl.when((s == 0) & (c == 0))
    def _():
        pltpu.sync_copy(acc_sh, acc0_hbm)
        pltpu.sync_copy(den_sh, den0_hbm)

    @pl.when((s == 0) & (c == 1))
    def _():
        pltpu.sync_copy(acc_sh, acc1_hbm)
        pltpu.sync_copy(den_sh, den1_hbm)


_pass2 = functools.partial(
    pl.kernel,
    compiler_params=_sc_params,
    out_type=[
        jax.ShapeDtypeStruct((N, H), f32),
        jax.ShapeDtypeStruct((N, H), f32),
        jax.ShapeDtypeStruct((N,), f32),
        jax.ShapeDtypeStruct((N,), f32),
    ],
    mesh=_mesh,
    scratch_types=[
        pltpu.VMEM((C,), jnp.int32),
        pltpu.VMEM((C,), jnp.int32),
        pltpu.VMEM((C, H), f32),
        pltpu.VMEM((C,), f32),
        pltpu.VMEM((C,), f32),
        pltpu.VMEM((C, H), f32),
        pltpu.VMEM((N,), f32),
        pltpu.VMEM((N,), f32),
        pltpu.VMEM((N,), f32),
        pltpu.VMEM((N,), f32),
        pltpu.VMEM((N,), f32),
        pltpu.VMEM_SHARED((N, H), f32),
        pltpu.VMEM_SHARED((N,), f32),
        pltpu.SemaphoreType.DMA,
    ],
)(_pass2_body)


def kernel(x, edge_index, edge_attr, params, Wlin, blin):
    src = edge_index[0].astype(jnp.int32)
    dst = edge_index[1].astype(jnp.int32)

    we3 = jnp.stack([p[4] for p in params])
    e3 = _edge_emb(edge_attr, we3)

    zn = jnp.zeros((N,), f32)
    znh = jnp.zeros((N, H), f32)

    wl, bl, wr, br, _, att, bias = params[0]
    xl, xr = _proj(x, wl, bl.reshape(1, H), wr, br.reshape(1, H))

    out = None
    for l in range(3):
        el = e3[l]
        alpha, s0, s1, c0, c1 = _pass1(src, dst, xl, xr, el, att, zn)
        a0, a1, d0, d1 = _pass2(src, dst, xl, alpha, s0, s1, c0, c1, zn, znh)
        d0 = d0.reshape(N, 1)
        d1 = d1.reshape(N, 1)
        if l < 2:
            nwl, nbl, nwr, nbr, _, natt, nbias = params[l + 1]
            xl, xr = _combine_proj(a0, a1, d0, d1, bias.reshape(1, H),
                                   nwl, nbl.reshape(1, H), nwr, nbr.reshape(1, H))
            att = natt
            bias = nbias
        else:
            out = _final(a0, a1, d0, d1, bias.reshape(1, H), Wlin,
                         blin.reshape(1, 1))
    return out


# speedup vs baseline: 13.4410x; 1.7960x over previous
import functools

import jax
import jax.numpy as jnp
from jax import lax
from jax.experimental import pallas as pl
from jax.experimental.pallas import tpu as pltpu
from jax.experimental.pallas import tpu_sc as plsc

N = 10000
E = 320000
D = 128
H = 16
DE = 16

NC = 2
NS = 16
NW = NC * NS
EPW = E // NW
C = 80
NCH = EPW // C
NG = C // 16

_mesh = plsc.VectorSubcoreMesh(
    core_axis_name="c", subcore_axis_name="s", num_cores=NC, num_subcores=NS)

f32 = jnp.float32


def _edge_emb_body(ea_ref, we_ref, out_ref):
    out_ref[0] = jnp.dot(ea_ref[...], we_ref[0], preferred_element_type=f32)


def _edge_emb(edge_attr, we3):
    EB = 4000
    return pl.pallas_call(
        _edge_emb_body,
        grid=(3, E // EB),
        in_specs=[
            pl.BlockSpec((EB, DE), lambda l, i: (i, 0)),
            pl.BlockSpec((1, DE, H), lambda l, i: (l, 0, 0)),
        ],
        out_specs=pl.BlockSpec((1, EB, H), lambda l, i: (l, i, 0)),
        out_shape=jax.ShapeDtypeStruct((3, E, H), f32),
    )(edge_attr, we3)


def _proj_body(x_ref, wl_ref, bl_ref, wr_ref, br_ref, xl_ref, xr_ref):
    xv = x_ref[...]
    xl_ref[...] = jnp.dot(xv, wl_ref[...], preferred_element_type=f32) + bl_ref[0]
    xr_ref[...] = jnp.dot(xv, wr_ref[...], preferred_element_type=f32) + br_ref[0]


def _proj(x, wl, bl, wr, br):
    NB = 2000
    din = x.shape[1]
    return pl.pallas_call(
        _proj_body,
        grid=(N // NB,),
        in_specs=[
            pl.BlockSpec((NB, din), lambda i: (i, 0)),
            pl.BlockSpec((din, H), lambda i: (0, 0)),
            pl.BlockSpec((1, H), lambda i: (0, 0)),
            pl.BlockSpec((din, H), lambda i: (0, 0)),
            pl.BlockSpec((1, H), lambda i: (0, 0)),
        ],
        out_specs=[
            pl.BlockSpec((NB, H), lambda i: (i, 0)),
            pl.BlockSpec((NB, H), lambda i: (i, 0)),
        ],
        out_shape=[
            jax.ShapeDtypeStruct((N, H), f32),
            jax.ShapeDtypeStruct((N, H), f32),
        ],
    )(x, wl, bl, wr, br)


def _combine_proj_body(a0_ref, a1_ref, d0_ref, d1_ref, bias_ref,
                       wl_ref, bl_ref, wr_ref, br_ref, xl_ref, xr_ref):
    acc = a0_ref[...] + a1_ref[...]
    den = d0_ref[...] + d1_ref[...] + 1e-16
    h = acc / den + bias_ref[0]
    xl_ref[...] = jnp.dot(h, wl_ref[...], preferred_element_type=f32) + bl_ref[0]
    xr_ref[...] = jnp.dot(h, wr_ref[...], preferred_element_type=f32) + br_ref[0]


def _combine_proj(a0, a1, d0, d1, bias, wl, bl, wr, br):
    NB = 2000
    return pl.pallas_call(
        _combine_proj_body,
        grid=(N // NB,),
        in_specs=[
            pl.BlockSpec((NB, H), lambda i: (i, 0)),
            pl.BlockSpec((NB, H), lambda i: (i, 0)),
            pl.BlockSpec((NB, 1), lambda i: (i, 0)),
            pl.BlockSpec((NB, 1), lambda i: (i, 0)),
            pl.BlockSpec((1, H), lambda i: (0, 0)),
            pl.BlockSpec((H, H), lambda i: (0, 0)),
            pl.BlockSpec((1, H), lambda i: (0, 0)),
            pl.BlockSpec((H, H), lambda i: (0, 0)),
            pl.BlockSpec((1, H), lambda i: (0, 0)),
        ],
        out_specs=[
            pl.BlockSpec((NB, H), lambda i: (i, 0)),
            pl.BlockSpec((NB, H), lambda i: (i, 0)),
        ],
        out_shape=[
            jax.ShapeDtypeStruct((N, H), f32),
            jax.ShapeDtypeStruct((N, H), f32),
        ],
    )(a0, a1, d0, d1, bias, wl, bl, wr, br)


def _final_body(a0_ref, a1_ref, d0_ref, d1_ref, bias_ref, wlin_ref, blin_ref,
                out_ref):
    i = pl.program_id(0)
    acc = a0_ref[...] + a1_ref[...]
    den = d0_ref[...] + d1_ref[...] + 1e-16
    h = acc / den + bias_ref[0]
    part = jnp.sum(jnp.dot(h, wlin_ref[...], preferred_element_type=f32))

    @pl.when(i == 0)
    def _():
        out_ref[...] = jnp.zeros((1, 1), f32)

    out_ref[...] += jnp.reshape(part / N, (1, 1))

    @pl.when(i == pl.num_programs(0) - 1)
    def _():
        out_ref[...] += blin_ref[...]


def _final(a0, a1, d0, d1, bias, wlin, blin):
    NB = 2000
    return pl.pallas_call(
        _final_body,
        grid=(N // NB,),
        in_specs=[
            pl.BlockSpec((NB, H), lambda i: (i, 0)),
            pl.BlockSpec((NB, H), lambda i: (i, 0)),
            pl.BlockSpec((NB, 1), lambda i: (i, 0)),
            pl.BlockSpec((NB, 1), lambda i: (i, 0)),
            pl.BlockSpec((1, H), lambda i: (0, 0)),
            pl.BlockSpec((H, 1), lambda i: (0, 0)),
            pl.BlockSpec((1, 1), lambda i: (0, 0)),
        ],
        out_specs=pl.BlockSpec((1, 1), lambda i: (0, 0)),
        out_shape=jax.ShapeDtypeStruct((1, 1), f32),
    )(a0, a1, d0, d1, bias, wlin, blin)


_sc_params = pltpu.CompilerParams(
    needs_layout_passes=False, use_tc_tiling_on_sc=False)


def _make_pass1(layer):
    with_cnt = layer == 0

    def body(src2_hbm, dst2_hbm, xl_hbm, xr_hbm, e3_hbm, att_hbm, zn_hbm, *rest):
        if with_cnt:
            (alpha_hbm, ssum0_hbm, ssum1_hbm, cnt0_hbm, cnt1_hbm,
             srcv2, dstv2, xlv0, xlv1, xrv0, xrv1, ev0, ev1, avb, onesv, attv,
             ssum_sh, cnt_sh, sgl0, sgl1, sgr0, sgr1, se0, se1, sss, ssc) = rest
        else:
            (alpha_hbm, ssum0_hbm, ssum1_hbm,
             srcv2, dstv2, xlv0, xlv1, xrv0, xrv1, ev0, ev1, avb, attv,
             ssum_sh, sgl0, sgl1, sgr0, sgr1, se0, se1, sss) = rest
            cnt_sh = onesv = ssc = None
        xlv = [xlv0, xlv1]
        xrv = [xrv0, xrv1]
        ev = [ev0, ev1]
        sgl = [sgl0, sgl1]
        sgr = [sgr0, sgr1]
        se = [se0, se1]

        c = lax.axis_index("c")
        s = lax.axis_index("s")
        wid = s * NC + c
        base0 = wid * EPW

        pltpu.sync_copy(att_hbm, attv)
        pltpu.sync_copy(src2_hbm.at[wid], srcv2)
        pltpu.sync_copy(dst2_hbm.at[wid], dstv2)

        if with_cnt:
            ones16 = jnp.full((16,), 1.0, f32)
            for b in range(NG):
                onesv[pl.ds(16 * b, 16)] = ones16

        @pl.when(s == 0)
        def _():
            pltpu.sync_copy(zn_hbm, ssum_sh)
            if with_cnt:
                pltpu.sync_copy(zn_hbm, cnt_sh)

        plsc.subcore_barrier()

        attvec = attv[...]
        attks = [attvec[k] for k in range(H)]
        iota16 = lax.iota(jnp.int32, 16)
        kvecs = [jnp.full((16,), k, jnp.int32) for k in range(H)]

        def issue_g(j, slot):
            pltpu.async_copy(xl_hbm.at[srcv2.at[j]], xlv[slot], sgl[slot])
            pltpu.async_copy(xr_hbm.at[dstv2.at[j]], xrv[slot], sgr[slot])
            pltpu.async_copy(e3_hbm.at[layer, pl.ds(base0 + j * C, C)],
                             ev[slot], se[slot])

        def process(j, slot):
            pltpu.make_async_copy(xl_hbm.at[srcv2.at[j]], xlv[slot], sgl[slot]).wait()
            pltpu.make_async_copy(xr_hbm.at[dstv2.at[j]], xrv[slot], sgr[slot]).wait()
            pltpu.make_async_copy(e3_hbm.at[layer, pl.ds(base0 + j * C, C)],
                                  ev[slot], se[slot]).wait()
            for b in range(NG):
                ivec = iota16 + b * 16
                acc = jnp.zeros((16,), f32)
                for k in range(H):
                    z = (plsc.load_gather(xlv[slot], [ivec, kvecs[k]])
                         + plsc.load_gather(xrv[slot], [ivec, kvecs[k]])
                         + plsc.load_gather(ev[slot], [ivec, kvecs[k]]))
                    m = jnp.maximum(z, 0.2 * z)
                    acc = acc + m * attks[k]
                avb[pl.ds(j * C + b * 16, 16)] = acc
            pltpu.async_copy(avb.at[pl.ds(j * C, C)], ssum_sh.at[dstv2.at[j]],
                             sss, add=True)
            if with_cnt:
                pltpu.async_copy(onesv, cnt_sh.at[dstv2.at[j]], ssc, add=True)

        issue_g(0, 0)
        issue_g(1, 1)
        process(0, 0)
        issue_g(2, 0)
        process(1, 1)

        def pair(t, carry):
            a = 2 * t
            issue_g(a + 1, 1)
            process(a, 0)
            issue_g(a + 2, 0)
            process(a + 1, 1)
            return carry

        lax.fori_loop(1, NCH // 2, pair, 0)
        process(NCH - 1, 0)

        def drain(i, carry):
            pltpu.make_async_copy(avb.at[pl.ds(0, C)], ssum_sh.at[dstv2.at[0]],
                                  sss).wait()
            if with_cnt:
                pltpu.make_async_copy(onesv, cnt_sh.at[dstv2.at[0]], ssc).wait()
            return carry

        lax.fori_loop(0, NCH, drain, 0)
        pltpu.sync_copy(avb, alpha_hbm.at[pl.ds(base0, EPW)])

        plsc.subcore_barrier()

        @pl.when((s == 0) & (c == 0))
        def _():
            pltpu.sync_copy(ssum_sh, ssum0_hbm)
            if with_cnt:
                pltpu.sync_copy(cnt_sh, cnt0_hbm)

        @pl.when((s == 0) & (c == 1))
        def _():
            pltpu.sync_copy(ssum_sh, ssum1_hbm)
            if with_cnt:
                pltpu.sync_copy(cnt_sh, cnt1_hbm)

    out_type = [jax.ShapeDtypeStruct((E,), f32),
                jax.ShapeDtypeStruct((N,), f32),
                jax.ShapeDtypeStruct((N,), f32)]
    scratch = [
        pltpu.VMEM((NCH, C), jnp.int32),
        pltpu.VMEM((NCH, C), jnp.int32),
        pltpu.VMEM((C, H), f32),
        pltpu.VMEM((C, H), f32),
        pltpu.VMEM((C, H), f32),
        pltpu.VMEM((C, H), f32),
        pltpu.VMEM((C, H), f32),
        pltpu.VMEM((C, H), f32),
        pltpu.VMEM((EPW,), f32),
    ]
    if with_cnt:
        out_type += [jax.ShapeDtypeStruct((N,), f32),
                     jax.ShapeDtypeStruct((N,), f32)]
        scratch.append(pltpu.VMEM((C,), f32))
    scratch.append(pltpu.VMEM((16,), f32))
    scratch.append(pltpu.VMEM_SHARED((N,), f32))
    if with_cnt:
        scratch.append(pltpu.VMEM_SHARED((N,), f32))
    scratch += [pltpu.SemaphoreType.DMA] * (8 if with_cnt else 7)

    return pl.kernel(body, out_type=out_type, mesh=_mesh,
                     scratch_types=scratch, compiler_params=_sc_params)


def _pass2_body(src2_hbm, dst2_hbm, xl_hbm, alpha_hbm,
                ssum0_hbm, ssum1_hbm, cnt0_hbm, cnt1_hbm, zn_hbm, znh_hbm,
                acc0_hbm, acc1_hbm, den0_hbm, den1_hbm,
                srcv2, dstv2, avb, exb, shift_v, t0v, t1v, c0v, c1v,
                xlv0, xlv1, rowv0, rowv1,
                acc_sh, den_sh, sgl0, sgl1, srs0, srs1, sds):
    xlv = [xlv0, xlv1]
    rowv = [rowv0, rowv1]
    sgl = [sgl0, sgl1]
    srs = [srs0, srs1]

    c = lax.axis_index("c")
    s = lax.axis_index("s")
    wid = s * NC + c
    base0 = wid * EPW

    pltpu.sync_copy(src2_hbm.at[wid], srcv2)
    pltpu.sync_copy(dst2_hbm.at[wid], dstv2)
    pltpu.sync_copy(alpha_hbm.at[pl.ds(base0, EPW)], avb)

    def issue_g(j, slot):
        pltpu.async_copy(xl_hbm.at[srcv2.at[j]], xlv[slot], sgl[slot])

    issue_g(0, 0)
    issue_g(1, 1)

    pltpu.sync_copy(ssum0_hbm, t0v)
    pltpu.sync_copy(ssum1_hbm, t1v)
    pltpu.sync_copy(cnt0_hbm, c0v)
    pltpu.sync_copy(cnt1_hbm, c1v)

    @pl.when(s == 0)
    def _():
        pltpu.sync_copy(znh_hbm, acc_sh)
        pltpu.sync_copy(zn_hbm, den_sh)

    def sbody(i, carry):
        sl = pl.ds(i * 16, 16)
        tot = t0v[sl] + t1v[sl]
        cnt = jnp.maximum(c0v[sl] + c1v[sl], 1.0)
        shift_v[sl] = tot / cnt
        return carry

    lax.fori_loop(0, N // 16, sbody, 0)

    plsc.subcore_barrier()

    def process(j, slot, first):
        pltpu.make_async_copy(xl_hbm.at[srcv2.at[j]], xlv[slot], sgl[slot]).wait()
        if not first:
            pltpu.make_async_copy(rowv[slot], acc_sh.at[dstv2.at[0]],
                                  srs[slot]).wait()
        for b in range(NG):
            sl = pl.ds(j * C + b * 16, 16)
            dvec = dstv2[j, pl.ds(b * 16, 16)]
            svec = plsc.load_gather(shift_v, [dvec])
            exvec = jnp.exp(avb[sl] - svec)
            exb[sl] = exvec
            for t in range(16):
                i = b * 16 + t
                rowv[slot][i, :] = xlv[slot][i, :] * exvec[t]
        pltpu.async_copy(exb.at[pl.ds(j * C, C)], den_sh.at[dstv2.at[j]],
                         sds, add=True)
        pltpu.async_copy(rowv[slot], acc_sh.at[dstv2.at[j]], srs[slot], add=True)

    process(0, 0, True)
    issue_g(2, 0)
    process(1, 1, True)

    def pair(t, carry):
        a = 2 * t
        issue_g(a + 1, 1)
        process(a, 0, False)
        issue_g(a + 2, 0)
        process(a + 1, 1, False)
        return carry

    lax.fori_loop(1, NCH // 2, pair, 0)
    process(NCH - 1, 0, False)

    def drain(i, carry):
        pltpu.make_async_copy(exb.at[pl.ds(0, C)], den_sh.at[dstv2.at[0]],
                              sds).wait()
        return carry

    lax.fori_loop(0, NCH, drain, 0)
    pltpu.make_async_copy(rowv[0], acc_sh.at[dstv2.at[0]], srs[0]).wait()
    pltpu.make_async_copy(rowv[1], acc_sh.at[dstv2.at[0]], srs[1]).wait()

    plsc.subcore_barrier()

    @pl.when((s == 0) & (c == 0))
    def _():
        pltpu.sync_copy(acc_sh, acc0_hbm)
        pltpu.sync_copy(den_sh, den0_hbm)

    @pl.when((s == 0) & (c == 1))
    def _():
        pltpu.sync_copy(acc_sh, acc1_hbm)
        pltpu.sync_copy(den_sh, den1_hbm)


_pass2 = pl.kernel(
    _pass2_body,
    out_type=[
        jax.ShapeDtypeStruct((N, H), f32),
        jax.ShapeDtypeStruct((N, H), f32),
        jax.ShapeDtypeStruct((N,), f32),
        jax.ShapeDtypeStruct((N,), f32),
    ],
    mesh=_mesh,
    scratch_types=[
        pltpu.VMEM((NCH, C), jnp.int32),
        pltpu.VMEM((NCH, C), jnp.int32),
        pltpu.VMEM((EPW,), f32),
        pltpu.VMEM((EPW,), f32),
        pltpu.VMEM((N,), f32),
        pltpu.VMEM((N,), f32),
        pltpu.VMEM((N,), f32),
        pltpu.VMEM((N,), f32),
        pltpu.VMEM((N,), f32),
        pltpu.VMEM((C, H), f32),
        pltpu.VMEM((C, H), f32),
        pltpu.VMEM((C, H), f32),
        pltpu.VMEM((C, H), f32),
        pltpu.VMEM_SHARED((N, H), f32),
        pltpu.VMEM_SHARED((N,), f32),
        pltpu.SemaphoreType.DMA,
        pltpu.SemaphoreType.DMA,
        pltpu.SemaphoreType.DMA,
        pltpu.SemaphoreType.DMA,
        pltpu.SemaphoreType.DMA,
    ],
    compiler_params=_sc_params,
)

_pass1_kernels = [_make_pass1(0), _make_pass1(1), _make_pass1(2)]


def kernel(x, edge_index, edge_attr, params, Wlin, blin):
    src = edge_index[0].astype(jnp.int32)
    dst = edge_index[1].astype(jnp.int32)
    src2 = src.reshape(NW, NCH, C)
    dst2 = dst.reshape(NW, NCH, C)

    we3 = jnp.stack([p[4] for p in params])
    e3 = _edge_emb(edge_attr, we3)

    zn = jnp.zeros((N,), f32)
    znh = jnp.zeros((N, H), f32)

    wl, bl, wr, br, _, att, bias = params[0]
    xl, xr = _proj(x, wl, bl.reshape(1, H), wr, br.reshape(1, H))

    out = None
    c0 = c1 = None
    for l in range(3):
        if l == 0:
            alpha, s0, s1, c0, c1 = _pass1_kernels[0](
                src2, dst2, xl, xr, e3, att, zn)
        else:
            alpha, s0, s1 = _pass1_kernels[l](src2, dst2, xl, xr, e3, att, zn)
        a0, a1, d0, d1 = _pass2(src2, dst2, xl, alpha, s0, s1, c0, c1,
                                zn, znh)
        d0 = d0.reshape(N, 1)
        d1 = d1.reshape(N, 1)
        if l < 2:
            nwl, nbl, nwr, nbr, _, natt, nbias = params[l + 1]
            xl, xr = _combine_proj(a0, a1, d0, d1, bias.reshape(1, H),
                                   nwl, nbl.reshape(1, H), nwr, nbr.reshape(1, H))
            att = natt
            bias = nbias
        else:
            out = _final(a0, a1, d0, d1, bias.reshape(1, H), Wlin,
                         blin.reshape(1, 1))
    return out
